# baseline (speedup 1.0000x reference)
import jax
import jax.numpy as jnp
from jax.experimental import pallas as pl
from jax.experimental.pallas import tpu as pltpu

_S = 2048
_D = 2048
_TN = 256
_TMS = 128
_TMAX = 21
_PAD = _TMAX * _TMS

_PAIR_A = (0, 0, 0, 1, 1, 2)
_PAIR_B = (1, 2, 3, 2, 3, 3)
_ALPHA = (0.4, 1.0, 0.85, 1.15)


def _route_kernel(ew_ref, pos_ref, coef_ref, meta_ref, mean_ref):
    v = ew_ref[:]
    cols = [v[:, e:e + 1] for e in range(4)]
    sel = []
    ws = []
    for e in range(4):
        rank = jnp.zeros_like(cols[e], dtype=jnp.int32)
        for f in range(4):
            if f == e:
                continue
            if f < e:
                beats = cols[f] >= cols[e]
            else:
                beats = cols[f] > cols[e]
            rank = rank + beats.astype(jnp.int32)
        s = (rank < 2).astype(jnp.float32)
        sel.append(s)
        ws.append(cols[e] * s)
    ssum = ws[0] + ws[1] + ws[2] + ws[3]
    inv = 1.0 / jnp.maximum(ssum, 1e-8)
    w = [wi * inv for wi in ws]
    s_w = w[0] + w[1] + w[2] + w[3]

    coef_ref[:, 0:1] = _ALPHA[0] * w[0]
    coef_ref[:, 1:2] = _ALPHA[1] * w[1]
    coef_ref[:, 2:3] = _ALPHA[2] * w[2]
    coef_ref[:, 3:4] = _ALPHA[3] * w[3]
    coef_ref[:, 4:5] = s_w
    coef_ref[:, 5:6] = 0.15 * w[2]
    coef_ref[:, 6:16] = jnp.zeros((_S, 10), jnp.float32)

    wcat = jnp.concatenate(w, axis=1)
    m = jnp.sum(wcat, axis=0, keepdims=True) / _S
    mrow = jnp.concatenate([m, jnp.zeros((1, 124), jnp.float32)], axis=1)
    mean_ref[:] = jnp.concatenate(
        [mrow, jnp.zeros((7, 128), jnp.float32)], axis=0)

    pairs = [sel[a] * sel[b] for a, b in zip(_PAIR_A, _PAIR_B)]
    ind = jnp.concatenate(pairs + [jnp.zeros((_S, 2), jnp.float32)], axis=1)

    row_i = jax.lax.broadcasted_iota(jnp.int32, (_S, _S), 0)
    col_i = jax.lax.broadcasted_iota(jnp.int32, (_S, _S), 1)
    ltri = (col_i < row_i).astype(jnp.float32).astype(jnp.bfloat16)
    ranks = jnp.dot(ltri, ind.astype(jnp.bfloat16),
                    preferred_element_type=jnp.float32)

    counts = jnp.sum(ind, axis=0, keepdims=True)
    pc = ((counts.astype(jnp.int32) + (_TMS - 1)) // _TMS) * _TMS
    offs = []
    acc = jnp.zeros((1, 1), jnp.int32)
    for p in range(6):
        offs.append(acc)
        acc = acc + pc[:, p:p + 1]
    total = acc
    off_row = jnp.concatenate(
        offs + [jnp.zeros((1, 2), jnp.int32)], axis=1).astype(jnp.float32)

    pos = jnp.sum(ind * (off_row + ranks), axis=1, keepdims=True)
    pos_ref[:] = pos.astype(jnp.int32)

    lane = jax.lax.broadcasted_iota(jnp.int32, (1, 32), 1)
    tp = jnp.zeros((1, 32), jnp.int32)
    for p in range(1, 6):
        tp = tp + (lane * _TMS >= offs[p]).astype(jnp.int32)
    num_tiles = total // _TMS
    meta_ref[:] = jnp.where(lane == 31, num_tiles, tp)


def _gmm_kernel(meta_ref, xc_ref, xm_ref, cs_ref,
                wc_ref, bc_ref, wb_ref, bb_ref,
                wr_ref, br_ref, wd_ref, bd_ref,
                out_ref, wbf_ref, acc_ref):
    n = pl.program_id(0)
    t = pl.program_id(1)
    w_refs = (wc_ref, wb_ref, wr_ref, wd_ref)
    b_refs = (bc_ref, bb_ref, br_ref, bd_ref)

    @pl.when(t == 0)
    def _cast_w():
        for e in range(4):
            wbf_ref[e] = w_refs[e][:].astype(jnp.bfloat16)

    num_tiles = meta_ref[31]

    @pl.when(t < num_tiles)
    def _tile():
        pw = meta_ref[t]
        xc = xc_ref[:]
        xm = xm_ref[:]
        acc_ref[:] = jnp.zeros((_TMS, _TN), jnp.float32)
        for e in range(4):
            in_pairs = [p for p in range(6)
                        if _PAIR_A[p] == e or _PAIR_B[p] == e]
            cond = ((pw == in_pairs[0]) | (pw == in_pairs[1])
                    | (pw == in_pairs[2]))

            @pl.when(cond)
            def _dot(e=e):
                h = jnp.dot(xc, wbf_ref[e, 0:_D, :],
                            preferred_element_type=jnp.float32)
                h = h + jnp.dot(xm, wbf_ref[e, _D:2 * _D, :],
                                preferred_element_type=jnp.float32)
                gate = jax.nn.sigmoid(h + b_refs[e][:])
                acc_ref[:] = acc_ref[:] + cs_ref[:, e:e + 1] * gate

        col = pl.ds(n * _TN, _TN)
        ctx32 = xc_ref[:, col].astype(jnp.float32)
        mem32 = xm_ref[:, col].astype(jnp.float32)
        out_ref[:] = (cs_ref[:, 4:5] * ctx32
                      + (mem32 - ctx32) * acc_ref[:]
                      + cs_ref[:, 5:6] * mem32)


def _route(ew):
    return pl.pallas_call(
        _route_kernel,
        out_shape=[jax.ShapeDtypeStruct((_S, 1), jnp.int32),
                   jax.ShapeDtypeStruct((_S, 16), jnp.float32),
                   jax.ShapeDtypeStruct((1, 32), jnp.int32),
                   jax.ShapeDtypeStruct((8, 128), jnp.float32)],
    )(ew)


def _gmm(meta, xc_s, xm_s, coef_s, weights, biases):
    n_tiles = _D // _TN
    in_specs = [pl.BlockSpec(memory_space=pltpu.SMEM),
                pl.BlockSpec((_TMS, _D), lambda n, t: (t, 0)),
                pl.BlockSpec((_TMS, _D), lambda n, t: (t, 0)),
                pl.BlockSpec((_TMS, 16), lambda n, t: (t, 0))]
    operands = [meta, xc_s, xm_s, coef_s]
    for W, b in zip(weights, biases):
        in_specs += [pl.BlockSpec((2 * _D, _TN), lambda n, t: (0, n)),
                     pl.BlockSpec((1, _TN), lambda n, t: (0, n))]
        operands += [W, b]
    return pl.pallas_call(
        _gmm_kernel,
        grid=(n_tiles, _TMAX),
        in_specs=in_specs,
        out_specs=pl.BlockSpec((_TMS, _TN), lambda n, t: (t, n)),
        out_shape=jax.ShapeDtypeStruct((_PAD, _D), jnp.float32),
        scratch_shapes=[pltpu.VMEM((4, 2 * _D, _TN), jnp.bfloat16),
                        pltpu.VMEM((_TMS, _TN), jnp.float32)],
        compiler_params=pltpu.CompilerParams(
            dimension_semantics=("arbitrary", "arbitrary")),
    )(*operands)


@jax.jit
def kernel(context_state, memory_state, expert_weights,
           W_conservative, b_conservative, W_base, b_base,
           W_bridge, b_bridge, W_memory_dominant, b_memory_dominant):
    B, S, d = context_state.shape
    ctx = context_state.reshape(S, d).astype(jnp.bfloat16)
    mem = memory_state.reshape(S, d).astype(jnp.bfloat16)
    ew = expert_weights.reshape(S, 4)
    biases = [b.reshape(1, d) for b in (b_conservative, b_base, b_bridge,
                                        b_memory_dominant)]
    weights = [W_conservative, W_base, W_bridge, W_memory_dominant]

    pos, coef, meta, mean_pad = _route(ew)
    p = pos[:, 0]

    xc_s = jnp.zeros((_PAD, d), jnp.bfloat16).at[p].set(ctx)
    xm_s = jnp.zeros((_PAD, d), jnp.bfloat16).at[p].set(mem)
    coef_s = jnp.zeros((_PAD, 16), jnp.float32).at[p].set(coef)

    out_sorted = _gmm(meta.reshape(32), xc_s, xm_s, coef_s, weights, biases)

    fused = out_sorted[p].reshape(B, S, d)
    mean_weights = mean_pad[0, 0:4]
    return fused, mean_weights
